# Initial kernel scaffold; baseline (speedup 1.0000x reference)
import jax
import jax.numpy as jnp
from jax.experimental import pallas as pl


def kernel(x, frag_idx, W1, b1, Wout, bout):
    raise NotImplementedError("write your pallas kernel here")


# speedup vs baseline: 2.7932x; 2.7932x over previous
import functools

import jax
import jax.numpy as jnp
from jax import lax
from jax.experimental import pallas as pl
from jax.experimental.pallas import tpu as pltpu
from jax.experimental.pallas import tpu_sc as plsc

N_ATOMS = 100000
N_FRAG = 200000
D_FEAT = 128
MID = 32
OUT_DIM = 3

NW = 32
CHUNK = 128
NPAD = 200704
RPW = NPAD // NW
NCH = RPW // CHUNK


def _tables_tc(x, G, bias96):
    BM = 2000

    def body(x_ref, g_ref, b_ref, p_ref, bb_ref):
        acc = jnp.dot(x_ref[...], g_ref[...], preferred_element_type=jnp.float32)
        acc = acc + b_ref[...]
        p_ref[...] = acc[:, :64]
        bb_ref[...] = acc[:, 64:]

    return pl.pallas_call(
        body,
        grid=(N_ATOMS // BM,),
        in_specs=[
            pl.BlockSpec((BM, D_FEAT), lambda i: (i, 0)),
            pl.BlockSpec((D_FEAT, 96), lambda i: (0, 0)),
            pl.BlockSpec((1, 96), lambda i: (0, 0)),
        ],
        out_specs=[
            pl.BlockSpec((BM, 64), lambda i: (i, 0)),
            pl.BlockSpec((BM, 32), lambda i: (i, 0)),
        ],
        out_shape=[
            jax.ShapeDtypeStruct((N_ATOMS, 64), jnp.float32),
            jax.ShapeDtypeStruct((N_ATOMS, 32), jnp.float32),
        ],
    )(x, G, bias96)


def _gather_sc(idx0, idx1, idx2, p64, b1t):
    mesh = plsc.VectorSubcoreMesh(
        core_axis_name="c", subcore_axis_name="s", num_cores=2, num_subcores=16
    )

    @functools.partial(
        pl.kernel,
        out_type=jax.ShapeDtypeStruct((NPAD, 64), jnp.float32),
        mesh=mesh,
        compiler_params=pltpu.CompilerParams(use_tc_tiling_on_sc=False),
        scratch_types=[
            pltpu.VMEM((NCH, CHUNK), jnp.int32),
            pltpu.VMEM((NCH, CHUNK), jnp.int32),
            pltpu.VMEM((NCH, CHUNK), jnp.int32),
            pltpu.VMEM((CHUNK, 64), jnp.float32),
            pltpu.VMEM((CHUNK, 64), jnp.float32),
            pltpu.VMEM((CHUNK, 64), jnp.float32),
            pltpu.VMEM((CHUNK, 64), jnp.float32),
            pltpu.VMEM((CHUNK, 32), jnp.float32),
            pltpu.VMEM((CHUNK, 32), jnp.float32),
            pltpu.VMEM((CHUNK, 64), jnp.float32),
            pltpu.VMEM((CHUNK, 64), jnp.float32),
            pltpu.SemaphoreType.DMA,
            pltpu.SemaphoreType.DMA,
            pltpu.SemaphoreType.DMA,
            pltpu.SemaphoreType.DMA,
        ],
    )
    def k(idx0_h, idx1_h, idx2_h, p_h, b_h, z_h,
          ib0, ib1, ib2, bp0a, bp0b, bp2a, bp2b, bba, bbb, zba, zbb,
          ga, gb, wa, wb):
        wid = lax.axis_index("s") * 2 + lax.axis_index("c")
        pltpu.sync_copy(idx0_h.at[wid], ib0)
        pltpu.sync_copy(idx1_h.at[wid], ib1)
        pltpu.sync_copy(idx2_h.at[wid], ib2)

        bp0 = (bp0a, bp0b)
        bp2 = (bp2a, bp2b)
        bb = (bba, bbb)
        zb = (zba, zbb)
        gsem = (ga, gb)
        wsem = (wa, wb)

        def issue(c, s):
            d0 = pltpu.async_copy(p_h.at[ib0.at[c]], bp0[s], gsem[s])
            d1 = pltpu.async_copy(b_h.at[ib1.at[c]], bb[s], gsem[s])
            d2 = pltpu.async_copy(p_h.at[ib2.at[c]], bp2[s], gsem[s])
            return (d0, d1, d2)

        def compute(s):
            p0r, p2r, bbr, zr = bp0[s], bp2[s], bb[s], zb[s]

            def row(r, _):
                for j in range(4):
                    v = (p0r[r, pl.ds(16 * j, 16)]
                         + bbr[r, pl.ds(16 * (j % 2), 16)]
                         + p2r[r, pl.ds(16 * ((j + 2) % 4), 16)])
                    zr[r, pl.ds(16 * j, 16)] = v
                return 0

            lax.fori_loop(0, CHUNK, row, 0)

        gd = [None, None]
        wd = [None, None]
        zbase = wid * RPW
        gd[0] = issue(0, 0)
        for c in range(NCH):
            s = c & 1
            if c + 1 < NCH:
                gd[1 - s] = issue(c + 1, 1 - s)
            for d in gd[s]:
                d.wait()
            if wd[s] is not None:
                wd[s].wait()
            compute(s)
            wd[s] = pltpu.async_copy(
                zb[s], z_h.at[pl.ds(zbase + c * CHUNK, CHUNK)], wsem[s]
            )
        for s in (0, 1):
            if wd[s] is not None:
                wd[s].wait()

    return k(idx0, idx1, idx2, p64, b1t)


def _readout_tc(zf, wbig, bias24):
    BM = 512
    nrows = zf.shape[0]

    def body(z_ref, w_ref, b_ref, o_ref):
        z = jnp.maximum(z_ref[...], 0.0)
        o_ref[...] = (
            jnp.dot(z, w_ref[...], preferred_element_type=jnp.float32) + b_ref[...]
        )

    return pl.pallas_call(
        body,
        grid=(nrows // BM,),
        in_specs=[
            pl.BlockSpec((BM, 512), lambda i: (i, 0)),
            pl.BlockSpec((512, 24), lambda i: (0, 0)),
            pl.BlockSpec((1, 24), lambda i: (0, 0)),
        ],
        out_specs=pl.BlockSpec((BM, 24), lambda i: (i, 0)),
        out_shape=jax.ShapeDtypeStruct((nrows, 24), jnp.float32),
    )(zf, wbig, bias24)


def kernel(x, frag_idx, W1, b1, Wout, bout):
    G = jnp.concatenate([W1[:D_FEAT], W1[2 * D_FEAT:], W1[D_FEAT:2 * D_FEAT]], axis=1)
    bias96 = jnp.concatenate([jnp.zeros((64,), jnp.float32), b1])[None, :]

    idx_pad = jnp.concatenate(
        [frag_idx, jnp.zeros((3, NPAD - N_FRAG), jnp.int32)], axis=1
    )
    idx2d = idx_pad.reshape(3, NW, NCH, CHUNK)

    p64, b1t = _tables_tc(x, G, bias96)
    z = _gather_sc(idx2d[0], idx2d[1], idx2d[2], p64, b1t)

    m = jnp.concatenate([Wout, Wout], axis=0)
    wbig = jnp.kron(jnp.eye(8, dtype=jnp.float32), m)
    bias24 = jnp.tile(bout, 8)[None, :]

    zf = z.reshape(NPAD // 8, 512)
    out = _readout_tc(zf, wbig, bias24)
    return out.reshape(NPAD, OUT_DIM)[:N_FRAG]
